# baseline (speedup 1.0000x reference)
import functools

import jax
import jax.numpy as jnp
from jax import lax
from jax.experimental import pallas as pl
from jax.experimental.pallas import tpu as pltpu
from jax.experimental.pallas import tpu_sc as plsc

N_OBJ, N_ROOM, N_ATTR, D = 10000, 1000, 2000, 128
NOP, NRP, NAP = 10240, 1024, 2048
E_OO, E_RO, E_AO = 320000, 40000, 60000
EP_OO, EP_RO, EP_AO = 327680, 40960, 65536
NC, NS, NT = 2, 16, 32
R_OO, R_RO, R_AO = EP_OO // NT // 128, EP_RO // NT // 128, EP_AO // NT // 128

_MESH = dict(core_axis_name="c", subcore_axis_name="s", num_cores=NC,
             num_subcores=NS)

_SP_OO = (32, 128)
_SP_RO = (4, 16)
_SP_AO = (8, 24)

_f32 = jnp.float32


def _sc_hists(srcoo, dstoo, srcro, dstro, srcao, dstao):
  bins = (NOP, NOP, NRP, NOP, NAP, NOP)
  rows = (R_OO, R_OO, R_RO, R_RO, R_AO, R_AO)
  out_type = [jax.ShapeDtypeStruct((NC, b, 16), _f32) for b in bins]
  scratch = [pltpu.VMEM((128, 128), jnp.int32),
             pltpu.VMEM((128, 16), _f32),
             pltpu.VMEM((128, 16), _f32)]
  scratch += [pltpu.VMEM_SHARED((b, 16), _f32) for b in bins]

  def body(s1, d1, s2, d2, s3, d3, o1, o2, o3, o4, o5, o6,
           idx_v, ones_v, zb_v, h1, h2, h3, h4, h5, h6):
    core = lax.axis_index("c")
    sid = lax.axis_index("s")
    wid = core * NS + sid
    ins = (s1, d1, s2, d2, s3, d3)
    outs = (o1, o2, o3, o4, o5, o6)
    hs = (h1, h2, h3, h4, h5, h6)

    @pl.loop(0, 128)
    def _(i):
      ones_v[i, :] = jnp.ones((16,), _f32)
      zb_v[i, :] = jnp.zeros((16,), _f32)

    for h, b in zip(hs, bins):
      rpt = b // NS
      base = sid * rpt
      nfull, rem = divmod(rpt, 128)
      for k in range(nfull):
        pltpu.sync_copy(zb_v, h.at[pl.ds(base + k * 128, 128)])
      if rem:
        pltpu.sync_copy(zb_v.at[pl.ds(0, rem)],
                        h.at[pl.ds(base + nfull * 128, rem)])
    plsc.subcore_barrier()

    for e2d, h, r in zip(ins, hs, rows):
      pltpu.sync_copy(e2d.at[wid], idx_v.at[pl.ds(0, r)])

      @pl.loop(0, r)
      def _(j):
        pltpu.sync_copy(ones_v, h.at[idx_v.at[j]], add=True)

    plsc.subcore_barrier()
    for h, o, b in zip(hs, outs, bins):
      rpt = b // NS
      pltpu.sync_copy(h.at[pl.ds(sid * rpt, rpt)],
                      o.at[core, pl.ds(sid * rpt, rpt)])

  f = pl.kernel(body, out_type=out_type,
                mesh=plsc.VectorSubcoreMesh(**_MESH),
                compiler_params=pltpu.CompilerParams(use_tc_tiling_on_sc=False),
                scratch_types=scratch)
  return f(srcoo, dstoo, srcro, dstro, srcao, dstao)


def _blk(r):
  for ib in (16, 12, 8, 6, 4, 2):
    if r % ib == 0:
      return ib
  raise ValueError(r)


def _sc_agg(tables, srcs, dsts, splits):
  nrel = len(tables)
  out_type = [jax.ShapeDtypeStruct((NC, NOP, D), _f32) for _ in range(nrel)]
  scratch = [pltpu.VMEM((16, 128), jnp.int32),
             pltpu.VMEM((16, 128), jnp.int32),
             pltpu.VMEM((128, D), _f32),
             pltpu.VMEM((128, D), _f32),
             pltpu.VMEM_SHARED((NOP, D), _f32),
             pltpu.SemaphoreType.DMA,
             pltpu.SemaphoreType.DMA]

  def body(*refs):
    args = refs[:3 * nrel]
    tbls, s2d, d2d = args[:nrel], args[nrel:2 * nrel], args[2 * nrel:3 * nrel]
    outs = refs[3 * nrel:4 * nrel]
    sidx, didx, rA, rB, acc, semA, semB = refs[4 * nrel:]
    core = lax.axis_index("c")
    sid = lax.axis_index("s")
    wid = core * NS + sid
    rpt = NOP // NS

    for p in range(nrel):
      tbl, s2, d2, o = tbls[p], s2d[p], d2d[p], outs[p]
      r0, r1 = splits[p]

      @pl.loop(0, 128)
      def _(i):
        for j in range(D // 16):
          rA[i, pl.ds(16 * j, 16)] = jnp.zeros((16,), _f32)

      for k in range(rpt // 128):
        pltpu.sync_copy(rA, acc.at[pl.ds(sid * rpt + k * 128, 128)])
      plsc.subcore_barrier()

      for core_val, r in ((0, r0), (1, r1)):
        base = (0 if core_val == 0 else NS * r0) + sid * r
        ib = _blk(r)

        @pl.when(core == core_val)
        def _():
          for b in range(r // ib):
            pltpu.sync_copy(s2.at[pl.ds(base + b * ib, ib)],
                            sidx.at[pl.ds(0, ib)])
            pltpu.sync_copy(d2.at[pl.ds(base + b * ib, ib)],
                            didx.at[pl.ds(0, ib)])

            pltpu.async_copy(tbl.at[sidx.at[0]], rA, semA)

            @pl.loop(0, ib, step=2)
            def _(i):
              pltpu.async_copy(tbl.at[sidx.at[i + 1]], rB, semB)
              pltpu.make_async_copy(tbl.at[sidx.at[i]], rA, semA).wait()
              pltpu.sync_copy(rA, acc.at[didx.at[i]], add=True)

              @pl.when(i + 2 < ib)
              def _():
                pltpu.async_copy(tbl.at[sidx.at[i + 2]], rA, semA)

              pltpu.make_async_copy(tbl.at[sidx.at[i + 1]], rB, semB).wait()
              pltpu.sync_copy(rB, acc.at[didx.at[i + 1]], add=True)

      plsc.subcore_barrier()
      pltpu.sync_copy(acc.at[pl.ds(sid * rpt, rpt)],
                      o.at[core, pl.ds(sid * rpt, rpt)])
      plsc.subcore_barrier()

  f = pl.kernel(body, out_type=out_type,
                mesh=plsc.VectorSubcoreMesh(**_MESH),
                compiler_params=pltpu.CompilerParams(use_tc_tiling_on_sc=False),
                scratch_types=scratch)
  return f(*tables, *srcs, *dsts)


def _scale_from(h):
  deg = h[0, :, 0:1] + h[1, :, 0:1]
  return lax.rsqrt(jnp.maximum(deg, 1.0))


def _tc_feat_body(xo, xr, xa, hso, hsr, hsa, fo, fr, fa):
  fo[...] = xo[...] * _scale_from(hso)
  fr[...] = xr[...] * _scale_from(hsr)
  fa[...] = xa[...] * _scale_from(hsa)


def _tc_feat(xo, xr, xa, hso, hsr, hsa):
  return pl.pallas_call(
      _tc_feat_body,
      out_shape=[jax.ShapeDtypeStruct((NOP, D), _f32),
                 jax.ShapeDtypeStruct((NRP, D), _f32),
                 jax.ShapeDtypeStruct((NAP, D), _f32)],
  )(xo, xr, xa, hso, hsr, hsa)


_BM = 1280


def _tc_mid_body(a1, a2, a3, hdo, hdr, hda, hso, Wi, bi, Wb, bb, f2):
  def gc(a, h, W, b):
    agg = (a[0] + a[1]) * _scale_from(h)
    r = jnp.dot(agg, W[...], preferred_element_type=_f32) + b[...]
    return jnp.maximum(r, 0.0)

  hmean = (gc(a1, hdo, Wi, bi) + gc(a2, hdr, Wi, bi)
           + gc(a3, hda, Wb, bb)) * (1.0 / 3.0)
  f2[...] = hmean * _scale_from(hso)


def _tc_mid(a1, a2, a3, hdo, hdr, hda, hso, Wi, bi, Wb, bb):
  blk_a = pl.BlockSpec((NC, _BM, D), lambda i: (0, i, 0))
  blk_h = pl.BlockSpec((NC, _BM, 16), lambda i: (0, i, 0))
  blk_w = pl.BlockSpec((D, D), lambda i: (0, 0))
  blk_b = pl.BlockSpec((1, D), lambda i: (0, 0))
  return pl.pallas_call(
      _tc_mid_body,
      grid=(NOP // _BM,),
      in_specs=[blk_a, blk_a, blk_a, blk_h, blk_h, blk_h, blk_h,
                blk_w, blk_b, blk_w, blk_b],
      out_specs=pl.BlockSpec((_BM, D), lambda i: (i, 0)),
      out_shape=jax.ShapeDtypeStruct((NOP, D), _f32),
  )(a1, a2, a3, hdo, hdr, hda, hso, Wi, bi, Wb, bb)


def _tc_final_body(a4, hdo, W2, b2, out, h2s):
  @pl.when(pl.program_id(0) == 0)
  def _():
    agg = a4[0, :N_OBJ, :] + a4[1, :N_OBJ, :]
    deg = hdo[0, :N_OBJ, 0:1] + hdo[1, :N_OBJ, 0:1]
    scaled = agg * lax.rsqrt(jnp.maximum(deg, 1.0))
    h2s[...] = jnp.dot(scaled, W2[...], preferred_element_type=_f32) + b2[...]

  out[0] = h2s[...]


def _tc_final(nf, a4, hdo, W2, b2):
  return pl.pallas_call(
      _tc_final_body,
      grid=(nf,),
      in_specs=[pl.BlockSpec((NC, NOP, D), lambda i: (0, 0, 0)),
                pl.BlockSpec((NC, NOP, 16), lambda i: (0, 0, 0)),
                pl.BlockSpec((D, D), lambda i: (0, 0)),
                pl.BlockSpec((1, D), lambda i: (0, 0))],
      out_specs=pl.BlockSpec((1, N_OBJ, D), lambda i: (i, 0, 0)),
      out_shape=jax.ShapeDtypeStruct((nf, N_OBJ, D), _f32),
      scratch_shapes=[pltpu.VMEM((N_OBJ, D), _f32)],
  )(a4, hdo, W2, b2)


def _pad_idx(e, ep, fill):
  return jnp.concatenate(
      [e, jnp.full((ep - e.shape[0],), fill, jnp.int32)]).reshape(NT, -1, 128)


def kernel(frames, x_object, x_room, x_attr, edge_oo, edge_ro_src,
           edge_ro_dst, edge_ao_src, edge_ao_dst, W_int, b_int, W_beh, b_beh,
           W2, b2):
  srcoo = _pad_idx(edge_oo[0], EP_OO, N_OBJ)
  dstoo = _pad_idx(edge_oo[1], EP_OO, N_OBJ)
  srcro = _pad_idx(edge_ro_src, EP_RO, N_ROOM)
  dstro = _pad_idx(edge_ro_dst, EP_RO, N_OBJ)
  srcao = _pad_idx(edge_ao_src, EP_AO, N_ATTR)
  dstao = _pad_idx(edge_ao_dst, EP_AO, N_OBJ)

  xo = jnp.pad(x_object, ((0, NOP - N_OBJ), (0, 0)))
  xr = jnp.pad(x_room, ((0, NRP - N_ROOM), (0, 0)))
  xa = jnp.pad(x_attr, ((0, NAP - N_ATTR), (0, 0)))
  bi, bb, b2r = (b.reshape(1, D) for b in (b_int, b_beh, b2))

  hso, hdo, hsr, hdr, hsa, hda = _sc_hists(srcoo, dstoo, srcro, dstro,
                                           srcao, dstao)
  fo, fr, fa = _tc_feat(xo, xr, xa, hso, hsr, hsa)
  flat = [a.reshape(-1, 128) for a in (srcoo, dstoo, srcro, dstro,
                                       srcao, dstao)]
  soo, doo, sro, dro, sao, dao = flat
  a1, a2, a3 = _sc_agg([fo, fr, fa], [soo, sro, sao], [doo, dro, dao],
                       [_SP_OO, _SP_RO, _SP_AO])
  f2 = _tc_mid(a1, a2, a3, hdo, hdr, hda, hso, W_int, bi, W_beh, bb)
  (a4,) = _sc_agg([f2], [soo], [doo], [_SP_OO])
  return _tc_final(frames.shape[0], a4, hdo, W2, b2r)


# speedup vs baseline: 1.2244x; 1.2244x over previous
import functools

import jax
import jax.numpy as jnp
from jax import lax
from jax.experimental import pallas as pl
from jax.experimental.pallas import tpu as pltpu
from jax.experimental.pallas import tpu_sc as plsc

N_OBJ, N_ROOM, N_ATTR, D = 10000, 1000, 2000, 128
NOP, NRP, NAP = 10240, 1024, 2048
E_OO, E_RO, E_AO = 320000, 40000, 60000
EP_OO, EP_RO, EP_AO = 327680, 40960, 65536
NC, NS, NT = 2, 16, 32
R_OO, R_RO, R_AO = EP_OO // NT // 128, EP_RO // NT // 128, EP_AO // NT // 128

_MESH = dict(core_axis_name="c", subcore_axis_name="s", num_cores=NC,
             num_subcores=NS)

_SP_OO = (128, 32)
_SP_RO = (16, 4)
_SP_AO = (24, 8)

_f32 = jnp.float32


def _sc_hists(srcoo, dstoo, srcro, dstro, srcao, dstao):
  bins = (NOP, NOP, NRP, NOP, NAP, NOP)
  rows = (R_OO, R_OO, R_RO, R_RO, R_AO, R_AO)
  out_type = [jax.ShapeDtypeStruct((NC, b, 16), _f32) for b in bins]
  scratch = [pltpu.VMEM((128, 128), jnp.int32),
             pltpu.VMEM((128, 16), _f32),
             pltpu.VMEM((128, 16), _f32)]
  scratch += [pltpu.VMEM_SHARED((b, 16), _f32) for b in bins]

  def body(s1, d1, s2, d2, s3, d3, o1, o2, o3, o4, o5, o6,
           idx_v, ones_v, zb_v, h1, h2, h3, h4, h5, h6):
    core = lax.axis_index("c")
    sid = lax.axis_index("s")
    wid = core * NS + sid
    ins = (s1, d1, s2, d2, s3, d3)
    outs = (o1, o2, o3, o4, o5, o6)
    hs = (h1, h2, h3, h4, h5, h6)

    @pl.loop(0, 128)
    def _(i):
      ones_v[i, :] = jnp.ones((16,), _f32)
      zb_v[i, :] = jnp.zeros((16,), _f32)

    for h, b in zip(hs, bins):
      rpt = b // NS
      base = sid * rpt
      nfull, rem = divmod(rpt, 128)
      for k in range(nfull):
        pltpu.sync_copy(zb_v, h.at[pl.ds(base + k * 128, 128)])
      if rem:
        pltpu.sync_copy(zb_v.at[pl.ds(0, rem)],
                        h.at[pl.ds(base + nfull * 128, rem)])
    plsc.subcore_barrier()

    for e2d, h, r in zip(ins, hs, rows):
      pltpu.sync_copy(e2d.at[wid], idx_v.at[pl.ds(0, r)])

      @pl.loop(0, r)
      def _(j):
        pltpu.sync_copy(ones_v, h.at[idx_v.at[j]], add=True)

    plsc.subcore_barrier()
    for h, o, b in zip(hs, outs, bins):
      rpt = b // NS
      pltpu.sync_copy(h.at[pl.ds(sid * rpt, rpt)],
                      o.at[core, pl.ds(sid * rpt, rpt)])

  f = pl.kernel(body, out_type=out_type,
                mesh=plsc.VectorSubcoreMesh(**_MESH),
                compiler_params=pltpu.CompilerParams(use_tc_tiling_on_sc=False),
                scratch_types=scratch)
  return f(srcoo, dstoo, srcro, dstro, srcao, dstao)


def _blk(r):
  for ib in (16, 12, 8, 6, 4, 2):
    if r % ib == 0:
      return ib
  raise ValueError(r)


def _sc_agg(tables, srcs, dsts, splits):
  nrel = len(tables)
  out_type = [jax.ShapeDtypeStruct((NC, NOP, D), _f32) for _ in range(nrel)]
  scratch = [pltpu.VMEM((16, 128), jnp.int32),
             pltpu.VMEM((16, 128), jnp.int32),
             pltpu.VMEM((128, D), _f32),
             pltpu.VMEM((128, D), _f32),
             pltpu.VMEM_SHARED((NOP, D), _f32),
             pltpu.SemaphoreType.DMA,
             pltpu.SemaphoreType.DMA]

  def body(*refs):
    args = refs[:3 * nrel]
    tbls, s2d, d2d = args[:nrel], args[nrel:2 * nrel], args[2 * nrel:3 * nrel]
    outs = refs[3 * nrel:4 * nrel]
    sidx, didx, rA, rB, acc, semA, semB = refs[4 * nrel:]
    core = lax.axis_index("c")
    sid = lax.axis_index("s")
    wid = core * NS + sid
    rpt = NOP // NS

    for p in range(nrel):
      tbl, s2, d2, o = tbls[p], s2d[p], d2d[p], outs[p]
      r0, r1 = splits[p]

      @pl.loop(0, 128)
      def _(i):
        for j in range(D // 16):
          rA[i, pl.ds(16 * j, 16)] = jnp.zeros((16,), _f32)

      for k in range(rpt // 128):
        pltpu.sync_copy(rA, acc.at[pl.ds(sid * rpt + k * 128, 128)])
      plsc.subcore_barrier()

      for core_val, r in ((0, r0), (1, r1)):
        base = (0 if core_val == 0 else NS * r0) + sid * r
        ib = _blk(r)

        @pl.when(core == core_val)
        def _():
          for b in range(r // ib):
            pltpu.sync_copy(s2.at[pl.ds(base + b * ib, ib)],
                            sidx.at[pl.ds(0, ib)])
            pltpu.sync_copy(d2.at[pl.ds(base + b * ib, ib)],
                            didx.at[pl.ds(0, ib)])

            pltpu.async_copy(tbl.at[sidx.at[0]], rA, semA)

            @pl.loop(0, ib, step=2)
            def _(i):
              pltpu.async_copy(tbl.at[sidx.at[i + 1]], rB, semB)
              pltpu.make_async_copy(tbl.at[sidx.at[i]], rA, semA).wait()
              pltpu.sync_copy(rA, acc.at[didx.at[i]], add=True)

              @pl.when(i + 2 < ib)
              def _():
                pltpu.async_copy(tbl.at[sidx.at[i + 2]], rA, semA)

              pltpu.make_async_copy(tbl.at[sidx.at[i + 1]], rB, semB).wait()
              pltpu.sync_copy(rB, acc.at[didx.at[i + 1]], add=True)

      plsc.subcore_barrier()
      pltpu.sync_copy(acc.at[pl.ds(sid * rpt, rpt)],
                      o.at[core, pl.ds(sid * rpt, rpt)])
      plsc.subcore_barrier()

  f = pl.kernel(body, out_type=out_type,
                mesh=plsc.VectorSubcoreMesh(**_MESH),
                compiler_params=pltpu.CompilerParams(use_tc_tiling_on_sc=False),
                scratch_types=scratch)
  return f(*tables, *srcs, *dsts)


def _scale_from(h):
  deg = h[0, :, 0:1] + h[1, :, 0:1]
  return lax.rsqrt(jnp.maximum(deg, 1.0))


def _tc_feat_body(xo, xr, xa, hso, hsr, hsa, fo, fr, fa):
  fo[...] = xo[...] * _scale_from(hso)
  fr[...] = xr[...] * _scale_from(hsr)
  fa[...] = xa[...] * _scale_from(hsa)


def _tc_feat(xo, xr, xa, hso, hsr, hsa):
  return pl.pallas_call(
      _tc_feat_body,
      out_shape=[jax.ShapeDtypeStruct((NOP, D), _f32),
                 jax.ShapeDtypeStruct((NRP, D), _f32),
                 jax.ShapeDtypeStruct((NAP, D), _f32)],
  )(xo, xr, xa, hso, hsr, hsa)


_BM = 1280


def _tc_mid_body(a1, a2, a3, hdo, hdr, hda, hso, Wi, bi, Wb, bb, f2):
  def gc(a, h, W, b):
    agg = (a[0] + a[1]) * _scale_from(h)
    r = jnp.dot(agg, W[...], preferred_element_type=_f32) + b[...]
    return jnp.maximum(r, 0.0)

  hmean = (gc(a1, hdo, Wi, bi) + gc(a2, hdr, Wi, bi)
           + gc(a3, hda, Wb, bb)) * (1.0 / 3.0)
  f2[...] = hmean * _scale_from(hso)


def _tc_mid(a1, a2, a3, hdo, hdr, hda, hso, Wi, bi, Wb, bb):
  blk_a = pl.BlockSpec((NC, _BM, D), lambda i: (0, i, 0))
  blk_h = pl.BlockSpec((NC, _BM, 16), lambda i: (0, i, 0))
  blk_w = pl.BlockSpec((D, D), lambda i: (0, 0))
  blk_b = pl.BlockSpec((1, D), lambda i: (0, 0))
  return pl.pallas_call(
      _tc_mid_body,
      grid=(NOP // _BM,),
      in_specs=[blk_a, blk_a, blk_a, blk_h, blk_h, blk_h, blk_h,
                blk_w, blk_b, blk_w, blk_b],
      out_specs=pl.BlockSpec((_BM, D), lambda i: (i, 0)),
      out_shape=jax.ShapeDtypeStruct((NOP, D), _f32),
  )(a1, a2, a3, hdo, hdr, hda, hso, Wi, bi, Wb, bb)


def _tc_final_body(a4, hdo, W2, b2, out, h2s):
  @pl.when(pl.program_id(0) == 0)
  def _():
    agg = a4[0, :N_OBJ, :] + a4[1, :N_OBJ, :]
    deg = hdo[0, :N_OBJ, 0:1] + hdo[1, :N_OBJ, 0:1]
    scaled = agg * lax.rsqrt(jnp.maximum(deg, 1.0))
    h2s[...] = jnp.dot(scaled, W2[...], preferred_element_type=_f32) + b2[...]

  out[0] = h2s[...]


def _tc_final(nf, a4, hdo, W2, b2):
  return pl.pallas_call(
      _tc_final_body,
      grid=(nf,),
      in_specs=[pl.BlockSpec((NC, NOP, D), lambda i: (0, 0, 0)),
                pl.BlockSpec((NC, NOP, 16), lambda i: (0, 0, 0)),
                pl.BlockSpec((D, D), lambda i: (0, 0)),
                pl.BlockSpec((1, D), lambda i: (0, 0))],
      out_specs=pl.BlockSpec((1, N_OBJ, D), lambda i: (i, 0, 0)),
      out_shape=jax.ShapeDtypeStruct((nf, N_OBJ, D), _f32),
      scratch_shapes=[pltpu.VMEM((N_OBJ, D), _f32)],
  )(a4, hdo, W2, b2)


def _pad_idx(e, ep, fill):
  return jnp.concatenate(
      [e, jnp.full((ep - e.shape[0],), fill, jnp.int32)]).reshape(NT, -1, 128)


def kernel(frames, x_object, x_room, x_attr, edge_oo, edge_ro_src,
           edge_ro_dst, edge_ao_src, edge_ao_dst, W_int, b_int, W_beh, b_beh,
           W2, b2):
  srcoo = _pad_idx(edge_oo[0], EP_OO, N_OBJ)
  dstoo = _pad_idx(edge_oo[1], EP_OO, N_OBJ)
  srcro = _pad_idx(edge_ro_src, EP_RO, N_ROOM)
  dstro = _pad_idx(edge_ro_dst, EP_RO, N_OBJ)
  srcao = _pad_idx(edge_ao_src, EP_AO, N_ATTR)
  dstao = _pad_idx(edge_ao_dst, EP_AO, N_OBJ)

  xo = jnp.pad(x_object, ((0, NOP - N_OBJ), (0, 0)))
  xr = jnp.pad(x_room, ((0, NRP - N_ROOM), (0, 0)))
  xa = jnp.pad(x_attr, ((0, NAP - N_ATTR), (0, 0)))
  bi, bb, b2r = (b.reshape(1, D) for b in (b_int, b_beh, b2))

  hso, hdo, hsr, hdr, hsa, hda = _sc_hists(srcoo, dstoo, srcro, dstro,
                                           srcao, dstao)
  fo, fr, fa = _tc_feat(xo, xr, xa, hso, hsr, hsa)
  flat = [a.reshape(-1, 128) for a in (srcoo, dstoo, srcro, dstro,
                                       srcao, dstao)]
  soo, doo, sro, dro, sao, dao = flat
  a1, a2, a3 = _sc_agg([fo, fr, fa], [soo, sro, sao], [doo, dro, dao],
                       [_SP_OO, _SP_RO, _SP_AO])
  f2 = _tc_mid(a1, a2, a3, hdo, hdr, hda, hso, W_int, bi, W_beh, bb)
  (a4,) = _sc_agg([f2], [soo], [doo], [_SP_OO])
  return _tc_final(frames.shape[0], a4, hdo, W2, b2r)
